# Initial kernel scaffold; baseline (speedup 1.0000x reference)
import jax
import jax.numpy as jnp
from jax.experimental import pallas as pl


def kernel(user_emb_table, item_emb_table, adj_rows, adj_cols, adj_vals, neg_rows, neg_cols, neg_vals):
    raise NotImplementedError("write your pallas kernel here")


# speedup vs baseline: 3.0897x; 3.0897x over previous
import functools

import jax
import jax.numpy as jnp
from jax import lax
from jax.experimental import pallas as pl
from jax.experimental.pallas import tpu as pltpu
from jax.experimental.pallas import tpu_sc as plsc

_N_USERS = 5000
_N = 10000
_D = 128
_NNZ = 320000
_NC = 2
_NS = 16
_NW = _NC * _NS
_CHUNK = 80
_NCHUNK = 128
_LANES = 16
_G = _D // _LANES
_ZROWS = 640


def _spmm_sc(rows3, cols3, vals3, table):
    mesh = plsc.VectorSubcoreMesh(core_axis_name="c", subcore_axis_name="s")

    @functools.partial(
        pl.kernel,
        out_type=jax.ShapeDtypeStruct((_NC, _N, _D), jnp.float32),
        mesh=mesh,
        scratch_types=dict(
            rows_b=pltpu.VMEM((4, _CHUNK), jnp.int32),
            cols_b=pltpu.VMEM((4, _CHUNK), jnp.int32),
            vals_b=pltpu.VMEM((4, _CHUNK), jnp.float32),
            gb0=pltpu.VMEM((_CHUNK, _D), jnp.float32),
            gb1=pltpu.VMEM((_CHUNK, _D), jnp.float32),
            sb0=pltpu.VMEM((_CHUNK, _D), jnp.float32),
            sb1=pltpu.VMEM((_CHUNK, _D), jnp.float32),
            acc=pltpu.VMEM_SHARED((_N, _D), jnp.float32),
            esem0=pltpu.SemaphoreType.DMA,
            esem1=pltpu.SemaphoreType.DMA,
            esem2=pltpu.SemaphoreType.DMA,
            esem3=pltpu.SemaphoreType.DMA,
            gsem0=pltpu.SemaphoreType.DMA,
            gsem1=pltpu.SemaphoreType.DMA,
            ssem0=pltpu.SemaphoreType.DMA,
            ssem1=pltpu.SemaphoreType.DMA,
        ),
    )
    def k(rows_hbm, cols_hbm, vals_hbm, table_hbm, out_hbm, *, rows_b,
          cols_b, vals_b, gb0, gb1, sb0, sb1, acc, esem0, esem1, esem2,
          esem3, gsem0, gsem1, ssem0, ssem1):
        cid = lax.axis_index("c")
        sid = lax.axis_index("s")
        wid = sid * _NC + cid
        gb = (gb0, gb1)
        sb = (sb0, sb1)
        esem = (esem0, esem1, esem2, esem3)
        gsem = (gsem0, gsem1)
        ssem = (ssem0, ssem1)

        def zrow(i, carry):
            for j in range(_G):
                sb0[i, pl.ds(j * _LANES, _LANES)] = jnp.zeros(
                    (_LANES,), jnp.float32)
            return carry

        lax.fori_loop(0, _CHUNK, zrow, 0)
        for kk in range(_ZROWS // _CHUNK):
            pltpu.sync_copy(
                sb0, acc.at[pl.ds(sid * 624 + kk * _CHUNK, _CHUNK)])
        plsc.subcore_barrier()

        def start_edges(c, slot):
            pltpu.async_copy(rows_hbm.at[wid, c], rows_b.at[slot],
                             esem[slot])
            pltpu.async_copy(cols_hbm.at[wid, c], cols_b.at[slot],
                             esem[slot])
            pltpu.async_copy(vals_hbm.at[wid, c], vals_b.at[slot],
                             esem[slot])

        def wait_edges(c, slot):
            pltpu.make_async_copy(rows_hbm.at[wid, c], rows_b.at[slot],
                                  esem[slot]).wait()
            pltpu.make_async_copy(cols_hbm.at[wid, c], cols_b.at[slot],
                                  esem[slot]).wait()
            pltpu.make_async_copy(vals_hbm.at[wid, c], vals_b.at[slot],
                                  esem[slot]).wait()

        def start_gather(slot, b):
            pltpu.async_copy(table_hbm.at[cols_b.at[slot]], gb[b], gsem[b])

        def wait_gather(slot, b):
            pltpu.make_async_copy(table_hbm.at[cols_b.at[slot]], gb[b],
                                  gsem[b]).wait()

        def start_scatter(slot, b):
            pltpu.async_copy(sb[b], acc.at[rows_b.at[slot]], ssem[b],
                             add=True)

        def wait_scatter(slot, b):
            pltpu.make_async_copy(sb[b], acc.at[rows_b.at[slot]],
                                  ssem[b]).wait()

        def scale(slot, b):
            @plsc.parallel_loop(0, _CHUNK // _LANES)
            def _(ii):
                i0 = ii * _LANES
                vv = vals_b[slot, pl.ds(i0, _LANES)]
                for e in range(_LANES):
                    vbc = jnp.broadcast_to(vv[e], (_LANES,))
                    for j in range(_G):
                        sl = pl.ds(j * _LANES, _LANES)
                        sb[b][i0 + e, sl] = gb[b][i0 + e, sl] * vbc

        for c in range(4):
            start_edges(c, c)
        wait_edges(0, 0)
        start_gather(0, 0)
        for m in range(2):
            wait_gather(m % 4, m % 2)
            wait_edges(m + 1, (m + 1) % 4)
            start_gather((m + 1) % 4, (m + 1) % 2)
            scale(m % 4, m % 2)
            start_scatter(m % 4, m % 2)

        def outer(o, carry):
            for b in range(4):
                m = 2 + o * 4 + b
                slot = (2 + b) % 4
                buf = b % 2
                wait_gather(slot, buf)
                wait_scatter(slot, buf)
                start_edges(m + 2, (slot + 2) % 4)
                wait_edges(m + 1, (slot + 1) % 4)
                start_gather((slot + 1) % 4, 1 - buf)
                scale(slot, buf)
                start_scatter(slot, buf)
            return carry

        lax.fori_loop(0, (_NCHUNK - 4) // 4, outer, 0)

        m0 = _NCHUNK - 2
        wait_gather(m0 % 4, 0)
        wait_scatter(m0 % 4, 0)
        wait_edges(m0 + 1, (m0 + 1) % 4)
        start_gather((m0 + 1) % 4, 1)
        scale(m0 % 4, 0)
        start_scatter(m0 % 4, 0)

        m1 = _NCHUNK - 1
        wait_gather(m1 % 4, 1)
        wait_scatter(m1 % 4, 1)
        scale(m1 % 4, 1)
        start_scatter(m1 % 4, 1)

        wait_scatter(m0 % 4, 0)
        wait_scatter(m1 % 4, 1)
        plsc.subcore_barrier()

        pltpu.sync_copy(acc.at[pl.ds(sid * 624, _ZROWS)],
                        out_hbm.at[cid, pl.ds(sid * 624, _ZROWS)])

    return k(rows3, cols3, vals3, table)


def _tc_add(a, b):
    def body(a_ref, b_ref, o_ref):
        o_ref[...] = a_ref[...] + b_ref[...]

    blk = pl.BlockSpec((1000, _D), lambda i: (i, 0))
    return pl.pallas_call(
        body,
        grid=(_N // 1000,),
        in_specs=[blk, blk],
        out_specs=blk,
        out_shape=jax.ShapeDtypeStruct((_N, _D), jnp.float32),
    )(a, b)


def _tc_assemble(pu, pi, qu0, qu1, qi0, qi1):
    def body(pu_r, pi_r, qu0_r, qu1_r, qi0_r, qi1_r, ou_r, oi_r):
        ou_r[:, : _D] = pu_r[...]
        ou_r[:, _D:] = qu0_r[...] + qu1_r[...]
        oi_r[:, : _D] = pi_r[...]
        oi_r[:, _D:] = qi0_r[...] + qi1_r[...]

    blk = pl.BlockSpec((1000, _D), lambda i: (i, 0))
    oblk = pl.BlockSpec((1000, 2 * _D), lambda i: (i, 0))
    osds = jax.ShapeDtypeStruct((_N_USERS, 2 * _D), jnp.float32)
    return pl.pallas_call(
        body,
        grid=(_N_USERS // 1000,),
        in_specs=[blk] * 6,
        out_specs=[oblk, oblk],
        out_shape=[osds, osds],
    )(pu, pi, qu0, qu1, qi0, qi1)


def _pad_edges(rows, cols, vals):
    pad = _NW * _NCHUNK * _CHUNK - _NNZ
    rows = jnp.pad(rows, (0, pad)).reshape(_NW, _NCHUNK, _CHUNK)
    cols = jnp.pad(cols, (0, pad)).reshape(_NW, _NCHUNK, _CHUNK)
    vals = jnp.pad(vals, (0, pad)).reshape(_NW, _NCHUNK, _CHUNK)
    return rows, cols, vals


def kernel(user_emb_table, item_emb_table, adj_rows, adj_cols, adj_vals,
           neg_rows, neg_cols, neg_vals):
    ego0 = jnp.concatenate([user_emb_table, item_emb_table], axis=0)
    ar, ac, av = _pad_edges(adj_rows, adj_cols, adj_vals)
    nr, nc, nv = _pad_edges(neg_rows, neg_cols, neg_vals)

    p = _spmm_sc(ar, ac, av, ego0)
    prop = _tc_add(p[0], p[1])
    q = _spmm_sc(nr, nc, nv, prop)

    user_all, item_all = _tc_assemble(
        prop[:_N_USERS], prop[_N_USERS:],
        q[0, :_N_USERS], q[1, :_N_USERS],
        q[0, _N_USERS:], q[1, _N_USERS:])
    return user_all, item_all


# speedup vs baseline: 3.4551x; 1.1183x over previous
import functools

import jax
import jax.numpy as jnp
from jax import lax
from jax.experimental import pallas as pl
from jax.experimental.pallas import tpu as pltpu
from jax.experimental.pallas import tpu_sc as plsc

_N_USERS = 5000
_N = 10000
_D = 128
_NNZ = 320000
_NC = 2
_NS = 16
_NW = _NC * _NS
_CHUNK = 80
_TOTC = 4096
_N0 = 176
_N1 = _TOTC // _NS - _N0
_LANES = 16
_G = _D // _LANES
_ZROWS = 640


def _spmm_sc(rows2, cols2, vals2, table):
    mesh = plsc.VectorSubcoreMesh(core_axis_name="c", subcore_axis_name="s")

    @functools.partial(
        pl.kernel,
        out_type=jax.ShapeDtypeStruct((_NC, _N, _D), jnp.float32),
        mesh=mesh,
        scratch_types=dict(
            rows_b=pltpu.VMEM((4, _CHUNK), jnp.int32),
            cols_b=pltpu.VMEM((4, _CHUNK), jnp.int32),
            vals_b=pltpu.VMEM((4, _CHUNK), jnp.float32),
            gb0=pltpu.VMEM((_CHUNK, _D), jnp.float32),
            gb1=pltpu.VMEM((_CHUNK, _D), jnp.float32),
            sb0=pltpu.VMEM((_CHUNK, _D), jnp.float32),
            sb1=pltpu.VMEM((_CHUNK, _D), jnp.float32),
            acc=pltpu.VMEM_SHARED((_N, _D), jnp.float32),
            esem0=pltpu.SemaphoreType.DMA,
            esem1=pltpu.SemaphoreType.DMA,
            esem2=pltpu.SemaphoreType.DMA,
            esem3=pltpu.SemaphoreType.DMA,
            gsem0=pltpu.SemaphoreType.DMA,
            gsem1=pltpu.SemaphoreType.DMA,
            ssem0=pltpu.SemaphoreType.DMA,
            ssem1=pltpu.SemaphoreType.DMA,
        ),
    )
    def k(rows_hbm, cols_hbm, vals_hbm, table_hbm, out_hbm, *, rows_b,
          cols_b, vals_b, gb0, gb1, sb0, sb1, acc, esem0, esem1, esem2,
          esem3, gsem0, gsem1, ssem0, ssem1):
        cid = lax.axis_index("c")
        sid = lax.axis_index("s")
        gb = (gb0, gb1)
        sb = (sb0, sb1)
        esem = (esem0, esem1, esem2, esem3)
        gsem = (gsem0, gsem1)
        ssem = (ssem0, ssem1)

        is0 = cid == 0
        base = jnp.where(is0, sid * _N0, _NS * _N0 + sid * _N1)
        nloc = jnp.where(is0, _N0, _N1)

        def zrow(i, carry):
            for j in range(_G):
                sb0[i, pl.ds(j * _LANES, _LANES)] = jnp.zeros(
                    (_LANES,), jnp.float32)
            return carry

        lax.fori_loop(0, _CHUNK, zrow, 0)
        for kk in range(_ZROWS // _CHUNK):
            pltpu.sync_copy(
                sb0, acc.at[pl.ds(sid * 624 + kk * _CHUNK, _CHUNK)])
        plsc.subcore_barrier()

        def start_edges(c, slot):
            pltpu.async_copy(rows_hbm.at[base + c], rows_b.at[slot],
                             esem[slot])
            pltpu.async_copy(cols_hbm.at[base + c], cols_b.at[slot],
                             esem[slot])
            pltpu.async_copy(vals_hbm.at[base + c], vals_b.at[slot],
                             esem[slot])

        def wait_edges(c, slot):
            pltpu.make_async_copy(rows_hbm.at[base + c], rows_b.at[slot],
                                  esem[slot]).wait()
            pltpu.make_async_copy(cols_hbm.at[base + c], cols_b.at[slot],
                                  esem[slot]).wait()
            pltpu.make_async_copy(vals_hbm.at[base + c], vals_b.at[slot],
                                  esem[slot]).wait()

        def start_gather(slot, b):
            pltpu.async_copy(table_hbm.at[cols_b.at[slot]], gb[b], gsem[b])

        def wait_gather(slot, b):
            pltpu.make_async_copy(table_hbm.at[cols_b.at[slot]], gb[b],
                                  gsem[b]).wait()

        def start_scatter(slot, b):
            pltpu.async_copy(sb[b], acc.at[rows_b.at[slot]], ssem[b],
                             add=True)

        def wait_scatter(slot, b):
            pltpu.make_async_copy(sb[b], acc.at[rows_b.at[slot]],
                                  ssem[b]).wait()

        def scale(slot, b):
            @plsc.parallel_loop(0, _CHUNK // _LANES)
            def _(ii):
                i0 = ii * _LANES
                vv = vals_b[slot, pl.ds(i0, _LANES)]
                for e in range(_LANES):
                    vbc = jnp.broadcast_to(vv[e], (_LANES,))
                    for j in range(_G):
                        sl = pl.ds(j * _LANES, _LANES)
                        sb[b][i0 + e, sl] = gb[b][i0 + e, sl] * vbc

        for c in range(4):
            start_edges(c, c)
        wait_edges(0, 0)
        start_gather(0, 0)
        for m in range(2):
            wait_gather(m % 4, m % 2)
            wait_edges(m + 1, (m + 1) % 4)
            start_gather((m + 1) % 4, (m + 1) % 2)
            scale(m % 4, m % 2)
            start_scatter(m % 4, m % 2)

        def outer(o, carry):
            for b in range(4):
                m = 2 + o * 4 + b
                slot = (2 + b) % 4
                buf = b % 2
                wait_gather(slot, buf)
                wait_scatter(slot, buf)
                start_edges(m + 2, (slot + 2) % 4)
                wait_edges(m + 1, (slot + 1) % 4)
                start_gather((slot + 1) % 4, 1 - buf)
                scale(slot, buf)
                start_scatter(slot, buf)
            return carry

        lax.fori_loop(0, (nloc - 4) // 4, outer, 0)

        m0 = nloc - 2
        wait_gather(2, 0)
        wait_scatter(2, 0)
        wait_edges(m0 + 1, 3)
        start_gather(3, 1)
        scale(2, 0)
        start_scatter(2, 0)

        wait_gather(3, 1)
        wait_scatter(3, 1)
        scale(3, 1)
        start_scatter(3, 1)

        wait_scatter(2, 0)
        wait_scatter(3, 1)
        plsc.subcore_barrier()

        pltpu.sync_copy(acc.at[pl.ds(sid * 624, _ZROWS)],
                        out_hbm.at[cid, pl.ds(sid * 624, _ZROWS)])

    return k(rows2, cols2, vals2, table)


def _tc_add(a, b):
    def body(a_ref, b_ref, o_ref):
        o_ref[...] = a_ref[...] + b_ref[...]

    blk = pl.BlockSpec((1000, _D), lambda i: (i, 0))
    return pl.pallas_call(
        body,
        grid=(_N // 1000,),
        in_specs=[blk, blk],
        out_specs=blk,
        out_shape=jax.ShapeDtypeStruct((_N, _D), jnp.float32),
    )(a, b)


def _tc_assemble(pu, pi, qu0, qu1, qi0, qi1):
    def body(pu_r, pi_r, qu0_r, qu1_r, qi0_r, qi1_r, ou_r, oi_r):
        ou_r[:, : _D] = pu_r[...]
        ou_r[:, _D:] = qu0_r[...] + qu1_r[...]
        oi_r[:, : _D] = pi_r[...]
        oi_r[:, _D:] = qi0_r[...] + qi1_r[...]

    blk = pl.BlockSpec((1000, _D), lambda i: (i, 0))
    oblk = pl.BlockSpec((1000, 2 * _D), lambda i: (i, 0))
    osds = jax.ShapeDtypeStruct((_N_USERS, 2 * _D), jnp.float32)
    return pl.pallas_call(
        body,
        grid=(_N_USERS // 1000,),
        in_specs=[blk] * 6,
        out_specs=[oblk, oblk],
        out_shape=[osds, osds],
    )(pu, pi, qu0, qu1, qi0, qi1)


def _pad_edges(rows, cols, vals):
    pad = _TOTC * _CHUNK - _NNZ
    rows = jnp.pad(rows, (0, pad)).reshape(_TOTC, _CHUNK)
    cols = jnp.pad(cols, (0, pad)).reshape(_TOTC, _CHUNK)
    vals = jnp.pad(vals, (0, pad)).reshape(_TOTC, _CHUNK)
    return rows, cols, vals


def kernel(user_emb_table, item_emb_table, adj_rows, adj_cols, adj_vals,
           neg_rows, neg_cols, neg_vals):
    ego0 = jnp.concatenate([user_emb_table, item_emb_table], axis=0)
    ar, ac, av = _pad_edges(adj_rows, adj_cols, adj_vals)
    nr, nc, nv = _pad_edges(neg_rows, neg_cols, neg_vals)

    p = _spmm_sc(ar, ac, av, ego0)
    prop = _tc_add(p[0], p[1])
    q = _spmm_sc(nr, nc, nv, prop)

    user_all, item_all = _tc_assemble(
        prop[:_N_USERS], prop[_N_USERS:],
        q[0, :_N_USERS], q[1, :_N_USERS],
        q[0, _N_USERS:], q[1, _N_USERS:])
    return user_all, item_all


# speedup vs baseline: 3.6229x; 1.0486x over previous
import functools

import jax
import jax.numpy as jnp
from jax import lax
from jax.experimental import pallas as pl
from jax.experimental.pallas import tpu as pltpu
from jax.experimental.pallas import tpu_sc as plsc

_N_USERS = 5000
_N = 10000
_D = 128
_NNZ = 320000
_NC = 2
_NS = 16
_NW = _NC * _NS
_CHUNK = 80
_TOTC = 4096
_N0 = 200
_N1 = _TOTC // _NS - _N0
_LANES = 16
_G = _D // _LANES
_ZROWS = 640


def _spmm_sc(rows2, cols2, vals2, table):
    mesh = plsc.VectorSubcoreMesh(core_axis_name="c", subcore_axis_name="s")

    @functools.partial(
        pl.kernel,
        out_type=jax.ShapeDtypeStruct((_NC, _N, _D), jnp.float32),
        mesh=mesh,
        scratch_types=dict(
            rows_b=pltpu.VMEM((4, _CHUNK), jnp.int32),
            cols_b=pltpu.VMEM((4, _CHUNK), jnp.int32),
            vals_b=pltpu.VMEM((4, _CHUNK), jnp.float32),
            gb0=pltpu.VMEM((_CHUNK, _D), jnp.float32),
            gb1=pltpu.VMEM((_CHUNK, _D), jnp.float32),
            sb0=pltpu.VMEM((_CHUNK, _D), jnp.float32),
            sb1=pltpu.VMEM((_CHUNK, _D), jnp.float32),
            acc=pltpu.VMEM_SHARED((_N, _D), jnp.float32),
            esem0=pltpu.SemaphoreType.DMA,
            esem1=pltpu.SemaphoreType.DMA,
            esem2=pltpu.SemaphoreType.DMA,
            esem3=pltpu.SemaphoreType.DMA,
            gsem0=pltpu.SemaphoreType.DMA,
            gsem1=pltpu.SemaphoreType.DMA,
            ssem0=pltpu.SemaphoreType.DMA,
            ssem1=pltpu.SemaphoreType.DMA,
        ),
    )
    def k(rows_hbm, cols_hbm, vals_hbm, table_hbm, out_hbm, *, rows_b,
          cols_b, vals_b, gb0, gb1, sb0, sb1, acc, esem0, esem1, esem2,
          esem3, gsem0, gsem1, ssem0, ssem1):
        cid = lax.axis_index("c")
        sid = lax.axis_index("s")
        gb = (gb0, gb1)
        sb = (sb0, sb1)
        esem = (esem0, esem1, esem2, esem3)
        gsem = (gsem0, gsem1)
        ssem = (ssem0, ssem1)

        is0 = cid == 0
        base = jnp.where(is0, sid * _N0, _NS * _N0 + sid * _N1)
        nloc = jnp.where(is0, _N0, _N1)

        def zrow(i, carry):
            for j in range(_G):
                sb0[i, pl.ds(j * _LANES, _LANES)] = jnp.zeros(
                    (_LANES,), jnp.float32)
            return carry

        lax.fori_loop(0, _CHUNK, zrow, 0)
        for kk in range(_ZROWS // _CHUNK):
            pltpu.sync_copy(
                sb0, acc.at[pl.ds(sid * 624 + kk * _CHUNK, _CHUNK)])
        plsc.subcore_barrier()

        def start_edges(c, slot):
            pltpu.async_copy(rows_hbm.at[base + c], rows_b.at[slot],
                             esem[slot])
            pltpu.async_copy(cols_hbm.at[base + c], cols_b.at[slot],
                             esem[slot])
            pltpu.async_copy(vals_hbm.at[base + c], vals_b.at[slot],
                             esem[slot])

        def wait_edges(c, slot):
            pltpu.make_async_copy(rows_hbm.at[base + c], rows_b.at[slot],
                                  esem[slot]).wait()
            pltpu.make_async_copy(cols_hbm.at[base + c], cols_b.at[slot],
                                  esem[slot]).wait()
            pltpu.make_async_copy(vals_hbm.at[base + c], vals_b.at[slot],
                                  esem[slot]).wait()

        def start_gather(slot, b):
            pltpu.async_copy(table_hbm.at[cols_b.at[slot]], gb[b], gsem[b])

        def wait_gather(slot, b):
            pltpu.make_async_copy(table_hbm.at[cols_b.at[slot]], gb[b],
                                  gsem[b]).wait()

        def start_scatter(slot, b):
            pltpu.async_copy(sb[b], acc.at[rows_b.at[slot]], ssem[b],
                             add=True)

        def wait_scatter(slot, b):
            pltpu.make_async_copy(sb[b], acc.at[rows_b.at[slot]],
                                  ssem[b]).wait()

        def scale(slot, b):
            @plsc.parallel_loop(0, _CHUNK // _LANES)
            def _(ii):
                i0 = ii * _LANES
                vv = vals_b[slot, pl.ds(i0, _LANES)]
                for e in range(_LANES):
                    vbc = jnp.broadcast_to(vv[e], (_LANES,))
                    for j in range(_G):
                        sl = pl.ds(j * _LANES, _LANES)
                        sb[b][i0 + e, sl] = gb[b][i0 + e, sl] * vbc

        for c in range(4):
            start_edges(c, c)
        wait_edges(0, 0)
        start_gather(0, 0)
        for m in range(2):
            wait_gather(m % 4, m % 2)
            wait_edges(m + 1, (m + 1) % 4)
            start_gather((m + 1) % 4, (m + 1) % 2)
            scale(m % 4, m % 2)
            start_scatter(m % 4, m % 2)

        def outer(o, carry):
            for b in range(4):
                m = 2 + o * 4 + b
                slot = (2 + b) % 4
                buf = b % 2
                wait_gather(slot, buf)
                wait_scatter(slot, buf)
                start_edges(m + 2, (slot + 2) % 4)
                wait_edges(m + 1, (slot + 1) % 4)
                start_gather((slot + 1) % 4, 1 - buf)
                scale(slot, buf)
                start_scatter(slot, buf)
            return carry

        lax.fori_loop(0, (nloc - 4) // 4, outer, 0)

        m0 = nloc - 2
        wait_gather(2, 0)
        wait_scatter(2, 0)
        wait_edges(m0 + 1, 3)
        start_gather(3, 1)
        scale(2, 0)
        start_scatter(2, 0)

        wait_gather(3, 1)
        wait_scatter(3, 1)
        scale(3, 1)
        start_scatter(3, 1)

        wait_scatter(2, 0)
        wait_scatter(3, 1)
        plsc.subcore_barrier()

        pltpu.sync_copy(acc.at[pl.ds(sid * 624, _ZROWS)],
                        out_hbm.at[cid, pl.ds(sid * 624, _ZROWS)])

    return k(rows2, cols2, vals2, table)


def _tc_add(a, b):
    def body(a_ref, b_ref, o_ref):
        o_ref[...] = a_ref[...] + b_ref[...]

    blk = pl.BlockSpec((1000, _D), lambda i: (i, 0))
    return pl.pallas_call(
        body,
        grid=(_N // 1000,),
        in_specs=[blk, blk],
        out_specs=blk,
        out_shape=jax.ShapeDtypeStruct((_N, _D), jnp.float32),
    )(a, b)


def _tc_assemble(pu, pi, qu0, qu1, qi0, qi1):
    def body(pu_r, pi_r, qu0_r, qu1_r, qi0_r, qi1_r, ou_r, oi_r):
        ou_r[:, : _D] = pu_r[...]
        ou_r[:, _D:] = qu0_r[...] + qu1_r[...]
        oi_r[:, : _D] = pi_r[...]
        oi_r[:, _D:] = qi0_r[...] + qi1_r[...]

    blk = pl.BlockSpec((1000, _D), lambda i: (i, 0))
    oblk = pl.BlockSpec((1000, 2 * _D), lambda i: (i, 0))
    osds = jax.ShapeDtypeStruct((_N_USERS, 2 * _D), jnp.float32)
    return pl.pallas_call(
        body,
        grid=(_N_USERS // 1000,),
        in_specs=[blk] * 6,
        out_specs=[oblk, oblk],
        out_shape=[osds, osds],
    )(pu, pi, qu0, qu1, qi0, qi1)


def _pad_edges(rows, cols, vals):
    pad = _TOTC * _CHUNK - _NNZ
    rows = jnp.pad(rows, (0, pad)).reshape(_TOTC, _CHUNK)
    cols = jnp.pad(cols, (0, pad)).reshape(_TOTC, _CHUNK)
    vals = jnp.pad(vals, (0, pad)).reshape(_TOTC, _CHUNK)
    return rows, cols, vals


def kernel(user_emb_table, item_emb_table, adj_rows, adj_cols, adj_vals,
           neg_rows, neg_cols, neg_vals):
    ego0 = jnp.concatenate([user_emb_table, item_emb_table], axis=0)
    ar, ac, av = _pad_edges(adj_rows, adj_cols, adj_vals)
    nr, nc, nv = _pad_edges(neg_rows, neg_cols, neg_vals)

    p = _spmm_sc(ar, ac, av, ego0)
    prop = _tc_add(p[0], p[1])
    q = _spmm_sc(nr, nc, nv, prop)

    user_all, item_all = _tc_assemble(
        prop[:_N_USERS], prop[_N_USERS:],
        q[0, :_N_USERS], q[1, :_N_USERS],
        q[0, _N_USERS:], q[1, _N_USERS:])
    return user_all, item_all


# speedup vs baseline: 5.7606x; 1.5900x over previous
import functools

import jax
import jax.numpy as jnp
from jax import lax
from jax.experimental import pallas as pl
from jax.experimental.pallas import tpu as pltpu
from jax.experimental.pallas import tpu_sc as plsc

_N_USERS = 5000
_N = 10000
_D = 128
_NNZ = 320000
_NC = 2
_NS = 16
_NW = _NC * _NS
_CHUNK = 80
_TOTC = 4096
_N0 = 200
_N1 = _TOTC // _NS - _N0
_LANES = 16
_G = _D // _LANES
_ZROWS = 640


def _spmm_sc(rows2, cols2, vals2, table):
    mesh = plsc.VectorSubcoreMesh(core_axis_name="c", subcore_axis_name="s")

    @functools.partial(
        pl.kernel,
        out_type=jax.ShapeDtypeStruct((_NC, _N, _D), jnp.float32),
        mesh=mesh,
        scratch_types=dict(
            rows_b=pltpu.VMEM((4, _CHUNK), jnp.int32),
            cols_b=pltpu.VMEM((4, _CHUNK), jnp.int32),
            vals_b=pltpu.VMEM((4, _CHUNK), jnp.float32),
            gb0=pltpu.VMEM((_CHUNK, _D), jnp.float32),
            gb1=pltpu.VMEM((_CHUNK, _D), jnp.float32),
            sb0=pltpu.VMEM((_CHUNK, _D), jnp.float32),
            sb1=pltpu.VMEM((_CHUNK, _D), jnp.float32),
            acc=pltpu.VMEM_SHARED((_N, _D), jnp.float32),
            esem0=pltpu.SemaphoreType.DMA,
            esem1=pltpu.SemaphoreType.DMA,
            esem2=pltpu.SemaphoreType.DMA,
            esem3=pltpu.SemaphoreType.DMA,
            gsem0=pltpu.SemaphoreType.DMA,
            gsem1=pltpu.SemaphoreType.DMA,
            ssem0=pltpu.SemaphoreType.DMA,
            ssem1=pltpu.SemaphoreType.DMA,
        ),
    )
    def k(rows_hbm, cols_hbm, vals_hbm, table_hbm, out_hbm, *, rows_b,
          cols_b, vals_b, gb0, gb1, sb0, sb1, acc, esem0, esem1, esem2,
          esem3, gsem0, gsem1, ssem0, ssem1):
        cid = lax.axis_index("c")
        sid = lax.axis_index("s")
        gb = (gb0, gb1)
        sb = (sb0, sb1)
        esem = (esem0, esem1, esem2, esem3)
        gsem = (gsem0, gsem1)
        ssem = (ssem0, ssem1)

        is0 = cid == 0
        base = jnp.where(is0, sid * _N0, _NS * _N0 + sid * _N1)
        nloc = jnp.where(is0, _N0, _N1)

        def zrow(i, carry):
            for j in range(_G):
                sb0[i, pl.ds(j * _LANES, _LANES)] = jnp.zeros(
                    (_LANES,), jnp.float32)
            return carry

        lax.fori_loop(0, _CHUNK, zrow, 0)
        for kk in range(_ZROWS // _CHUNK):
            pltpu.sync_copy(
                sb0, acc.at[pl.ds(sid * 624 + kk * _CHUNK, _CHUNK)])
        plsc.subcore_barrier()

        def start_edges(c, slot):
            pltpu.async_copy(rows_hbm.at[base + c], rows_b.at[slot],
                             esem[slot])
            pltpu.async_copy(cols_hbm.at[base + c], cols_b.at[slot],
                             esem[slot])
            pltpu.async_copy(vals_hbm.at[base + c], vals_b.at[slot],
                             esem[slot])

        def wait_edges(c, slot):
            pltpu.make_async_copy(rows_hbm.at[base + c], rows_b.at[slot],
                                  esem[slot]).wait()
            pltpu.make_async_copy(cols_hbm.at[base + c], cols_b.at[slot],
                                  esem[slot]).wait()
            pltpu.make_async_copy(vals_hbm.at[base + c], vals_b.at[slot],
                                  esem[slot]).wait()

        def start_gather(slot, b):
            pltpu.async_copy(table_hbm.at[cols_b.at[slot]], gb[b], gsem[b])

        def wait_gather(slot, b):
            pltpu.make_async_copy(table_hbm.at[cols_b.at[slot]], gb[b],
                                  gsem[b]).wait()

        def start_scatter(slot, b):
            pltpu.async_copy(sb[b], acc.at[rows_b.at[slot]], ssem[b],
                             add=True)

        def wait_scatter(slot, b):
            pltpu.make_async_copy(sb[b], acc.at[rows_b.at[slot]],
                                  ssem[b]).wait()

        def scale(slot, b):
            @plsc.parallel_loop(0, _CHUNK // _LANES)
            def _(ii):
                i0 = ii * _LANES
                vv = vals_b[slot, pl.ds(i0, _LANES)]
                for e in range(_LANES):
                    vbc = jnp.broadcast_to(vv[e], (_LANES,))
                    for j in range(_G):
                        sl = pl.ds(j * _LANES, _LANES)
                        sb[b][i0 + e, sl] = gb[b][i0 + e, sl] * vbc

        for c in range(4):
            start_edges(c, c)
        wait_edges(0, 0)
        start_gather(0, 0)
        for m in range(2):
            wait_gather(m % 4, m % 2)
            wait_edges(m + 1, (m + 1) % 4)
            start_gather((m + 1) % 4, (m + 1) % 2)
            scale(m % 4, m % 2)
            start_scatter(m % 4, m % 2)

        def outer(o, carry):
            for b in range(4):
                m = 2 + o * 4 + b
                slot = (2 + b) % 4
                buf = b % 2
                wait_gather(slot, buf)
                wait_scatter(slot, buf)
                start_edges(m + 2, (slot + 2) % 4)
                wait_edges(m + 1, (slot + 1) % 4)
                start_gather((slot + 1) % 4, 1 - buf)
                scale(slot, buf)
                start_scatter(slot, buf)
            return carry

        lax.fori_loop(0, (nloc - 4) // 4, outer, 0)

        m0 = nloc - 2
        wait_gather(2, 0)
        wait_scatter(2, 0)
        wait_edges(m0 + 1, 3)
        start_gather(3, 1)
        scale(2, 0)
        start_scatter(2, 0)

        wait_gather(3, 1)
        wait_scatter(3, 1)
        scale(3, 1)
        start_scatter(3, 1)

        wait_scatter(2, 0)
        wait_scatter(3, 1)
        plsc.subcore_barrier()

        pltpu.sync_copy(acc.at[pl.ds(sid * 624, _ZROWS)],
                        out_hbm.at[cid, pl.ds(sid * 624, _ZROWS)])

    return k(rows2, cols2, vals2, table)


def _tc_add(a, b):
    def body(a_ref, b_ref, o_ref):
        o_ref[...] = a_ref[...] + b_ref[...]

    blk = pl.BlockSpec((1000, _D), lambda i: (i, 0))
    return pl.pallas_call(
        body,
        grid=(_N // 1000,),
        in_specs=[blk, blk],
        out_specs=blk,
        out_shape=jax.ShapeDtypeStruct((_N, _D), jnp.float32),
    )(a, b)


def _tc_assemble(pu, pi, qu0, qu1, qi0, qi1):
    def body(pu_r, pi_r, qu0_r, qu1_r, qi0_r, qi1_r, ou_r, oi_r):
        ou_r[:, : _D] = pu_r[...]
        ou_r[:, _D:] = qu0_r[...] + qu1_r[...]
        oi_r[:, : _D] = pi_r[...]
        oi_r[:, _D:] = qi0_r[...] + qi1_r[...]

    blk = pl.BlockSpec((1000, _D), lambda i: (i, 0))
    oblk = pl.BlockSpec((1000, 2 * _D), lambda i: (i, 0))
    osds = jax.ShapeDtypeStruct((_N_USERS, 2 * _D), jnp.float32)
    return pl.pallas_call(
        body,
        grid=(_N_USERS // 1000,),
        in_specs=[blk] * 6,
        out_specs=[oblk, oblk],
        out_shape=[osds, osds],
    )(pu, pi, qu0, qu1, qi0, qi1)


def _pad_edges(rows, cols, vals):
    pad = _TOTC * _CHUNK - _NNZ
    spread = (jnp.arange(pad, dtype=jnp.int32) * 37) % _N
    rows = jnp.concatenate([rows, spread]).reshape(_TOTC, _CHUNK)
    cols = jnp.concatenate([cols, spread]).reshape(_TOTC, _CHUNK)
    vals = jnp.pad(vals, (0, pad)).reshape(_TOTC, _CHUNK)
    return rows, cols, vals


def kernel(user_emb_table, item_emb_table, adj_rows, adj_cols, adj_vals,
           neg_rows, neg_cols, neg_vals):
    ego0 = jnp.concatenate([user_emb_table, item_emb_table], axis=0)
    ar, ac, av = _pad_edges(adj_rows, adj_cols, adj_vals)
    nr, nc, nv = _pad_edges(neg_rows, neg_cols, neg_vals)

    p = _spmm_sc(ar, ac, av, ego0)
    prop = _tc_add(p[0], p[1])
    q = _spmm_sc(nr, nc, nv, prop)

    user_all, item_all = _tc_assemble(
        prop[:_N_USERS], prop[_N_USERS:],
        q[0, :_N_USERS], q[1, :_N_USERS],
        q[0, _N_USERS:], q[1, _N_USERS:])
    return user_all, item_all


# speedup vs baseline: 8.0557x; 1.3984x over previous
import functools

import jax
import jax.numpy as jnp
from jax import lax
from jax.experimental import pallas as pl
from jax.experimental.pallas import tpu as pltpu
from jax.experimental.pallas import tpu_sc as plsc

_N_USERS = 5000
_N = 10000
_D = 128
_NNZ = 320000
_NC = 2
_NS = 16
_NW = _NC * _NS
_CHUNK = 80
_TOTC = 4096
_N0 = 128
_N1 = _TOTC // _NS - _N0
_LANES = 16
_G = _D // _LANES
_ZROWS = 640


def _spmm_sc(rows2, cols2, vals2, table):
    mesh = plsc.VectorSubcoreMesh(core_axis_name="c", subcore_axis_name="s")

    @functools.partial(
        pl.kernel,
        out_type=jax.ShapeDtypeStruct((_NC, _N, _D), jnp.float32),
        mesh=mesh,
        scratch_types=dict(
            rows_b=pltpu.VMEM((4, _CHUNK), jnp.int32),
            cols_b=pltpu.VMEM((4, _CHUNK), jnp.int32),
            vals_b=pltpu.VMEM((4, _CHUNK), jnp.float32),
            gb0=pltpu.VMEM((_CHUNK, _D), jnp.float32),
            gb1=pltpu.VMEM((_CHUNK, _D), jnp.float32),
            sb0=pltpu.VMEM((_CHUNK, _D), jnp.float32),
            sb1=pltpu.VMEM((_CHUNK, _D), jnp.float32),
            acc=pltpu.VMEM_SHARED((_N, _D), jnp.float32),
            esem0=pltpu.SemaphoreType.DMA,
            esem1=pltpu.SemaphoreType.DMA,
            esem2=pltpu.SemaphoreType.DMA,
            esem3=pltpu.SemaphoreType.DMA,
            gsem0=pltpu.SemaphoreType.DMA,
            gsem1=pltpu.SemaphoreType.DMA,
            ssem0=pltpu.SemaphoreType.DMA,
            ssem1=pltpu.SemaphoreType.DMA,
        ),
    )
    def k(rows_hbm, cols_hbm, vals_hbm, table_hbm, out_hbm, *, rows_b,
          cols_b, vals_b, gb0, gb1, sb0, sb1, acc, esem0, esem1, esem2,
          esem3, gsem0, gsem1, ssem0, ssem1):
        cid = lax.axis_index("c")
        sid = lax.axis_index("s")
        gb = (gb0, gb1)
        sb = (sb0, sb1)
        esem = (esem0, esem1, esem2, esem3)
        gsem = (gsem0, gsem1)
        ssem = (ssem0, ssem1)

        is0 = cid == 0
        base = jnp.where(is0, sid * _N0, _NS * _N0 + sid * _N1)
        nloc = jnp.where(is0, _N0, _N1)

        def zrow(i, carry):
            for j in range(_G):
                sb0[i, pl.ds(j * _LANES, _LANES)] = jnp.zeros(
                    (_LANES,), jnp.float32)
            return carry

        lax.fori_loop(0, _CHUNK, zrow, 0)
        for kk in range(_ZROWS // _CHUNK):
            pltpu.sync_copy(
                sb0, acc.at[pl.ds(sid * 624 + kk * _CHUNK, _CHUNK)])
        plsc.subcore_barrier()

        def start_edges(c, slot):
            pltpu.async_copy(rows_hbm.at[base + c], rows_b.at[slot],
                             esem[slot])
            pltpu.async_copy(cols_hbm.at[base + c], cols_b.at[slot],
                             esem[slot])
            pltpu.async_copy(vals_hbm.at[base + c], vals_b.at[slot],
                             esem[slot])

        def wait_edges(c, slot):
            pltpu.make_async_copy(rows_hbm.at[base + c], rows_b.at[slot],
                                  esem[slot]).wait()
            pltpu.make_async_copy(cols_hbm.at[base + c], cols_b.at[slot],
                                  esem[slot]).wait()
            pltpu.make_async_copy(vals_hbm.at[base + c], vals_b.at[slot],
                                  esem[slot]).wait()

        def start_gather(slot, b):
            pltpu.async_copy(table_hbm.at[cols_b.at[slot]], gb[b], gsem[b])

        def wait_gather(slot, b):
            pltpu.make_async_copy(table_hbm.at[cols_b.at[slot]], gb[b],
                                  gsem[b]).wait()

        def start_scatter(slot, b):
            pltpu.async_copy(sb[b], acc.at[rows_b.at[slot]], ssem[b],
                             add=True)

        def wait_scatter(slot, b):
            pltpu.make_async_copy(sb[b], acc.at[rows_b.at[slot]],
                                  ssem[b]).wait()

        def scale(slot, b):
            @plsc.parallel_loop(0, _CHUNK // _LANES)
            def _(ii):
                i0 = ii * _LANES
                vv = vals_b[slot, pl.ds(i0, _LANES)]
                for e in range(_LANES):
                    vbc = jnp.broadcast_to(vv[e], (_LANES,))
                    for j in range(_G):
                        sl = pl.ds(j * _LANES, _LANES)
                        sb[b][i0 + e, sl] = gb[b][i0 + e, sl] * vbc

        for c in range(4):
            start_edges(c, c)
        wait_edges(0, 0)
        start_gather(0, 0)
        for m in range(2):
            wait_gather(m % 4, m % 2)
            wait_edges(m + 1, (m + 1) % 4)
            start_gather((m + 1) % 4, (m + 1) % 2)
            scale(m % 4, m % 2)
            start_scatter(m % 4, m % 2)

        def outer(o, carry):
            for b in range(4):
                m = 2 + o * 4 + b
                slot = (2 + b) % 4
                buf = b % 2
                wait_gather(slot, buf)
                wait_scatter(slot, buf)
                start_edges(m + 2, (slot + 2) % 4)
                wait_edges(m + 1, (slot + 1) % 4)
                start_gather((slot + 1) % 4, 1 - buf)
                scale(slot, buf)
                start_scatter(slot, buf)
            return carry

        lax.fori_loop(0, (nloc - 4) // 4, outer, 0)

        m0 = nloc - 2
        wait_gather(2, 0)
        wait_scatter(2, 0)
        wait_edges(m0 + 1, 3)
        start_gather(3, 1)
        scale(2, 0)
        start_scatter(2, 0)

        wait_gather(3, 1)
        wait_scatter(3, 1)
        scale(3, 1)
        start_scatter(3, 1)

        wait_scatter(2, 0)
        wait_scatter(3, 1)
        plsc.subcore_barrier()

        pltpu.sync_copy(acc.at[pl.ds(sid * 624, _ZROWS)],
                        out_hbm.at[cid, pl.ds(sid * 624, _ZROWS)])

    return k(rows2, cols2, vals2, table)


def _tc_add(a, b):
    def body(a_ref, b_ref, o_ref):
        o_ref[...] = a_ref[...] + b_ref[...]

    blk = pl.BlockSpec((1000, _D), lambda i: (i, 0))
    return pl.pallas_call(
        body,
        grid=(_N // 1000,),
        in_specs=[blk, blk],
        out_specs=blk,
        out_shape=jax.ShapeDtypeStruct((_N, _D), jnp.float32),
    )(a, b)


def _tc_assemble(pu, pi, qu0, qu1, qi0, qi1):
    def body(pu_r, pi_r, qu0_r, qu1_r, qi0_r, qi1_r, ou_r, oi_r):
        ou_r[:, : _D] = pu_r[...]
        ou_r[:, _D:] = qu0_r[...] + qu1_r[...]
        oi_r[:, : _D] = pi_r[...]
        oi_r[:, _D:] = qi0_r[...] + qi1_r[...]

    blk = pl.BlockSpec((1000, _D), lambda i: (i, 0))
    oblk = pl.BlockSpec((1000, 2 * _D), lambda i: (i, 0))
    osds = jax.ShapeDtypeStruct((_N_USERS, 2 * _D), jnp.float32)
    return pl.pallas_call(
        body,
        grid=(_N_USERS // 1000,),
        in_specs=[blk] * 6,
        out_specs=[oblk, oblk],
        out_shape=[osds, osds],
    )(pu, pi, qu0, qu1, qi0, qi1)


def _pad_edges(rows, cols, vals):
    pad = _TOTC * _CHUNK - _NNZ
    spread = (jnp.arange(pad, dtype=jnp.int32) * 37) % _N
    rows = jnp.concatenate([rows, spread]).reshape(_TOTC, _CHUNK)
    cols = jnp.concatenate([cols, spread]).reshape(_TOTC, _CHUNK)
    vals = jnp.pad(vals, (0, pad)).reshape(_TOTC, _CHUNK)
    return rows, cols, vals


def kernel(user_emb_table, item_emb_table, adj_rows, adj_cols, adj_vals,
           neg_rows, neg_cols, neg_vals):
    ego0 = jnp.concatenate([user_emb_table, item_emb_table], axis=0)
    ar, ac, av = _pad_edges(adj_rows, adj_cols, adj_vals)
    nr, nc, nv = _pad_edges(neg_rows, neg_cols, neg_vals)

    p = _spmm_sc(ar, ac, av, ego0)
    prop = _tc_add(p[0], p[1])
    q = _spmm_sc(nr, nc, nv, prop)

    user_all, item_all = _tc_assemble(
        prop[:_N_USERS], prop[_N_USERS:],
        q[0, :_N_USERS], q[1, :_N_USERS],
        q[0, _N_USERS:], q[1, _N_USERS:])
    return user_all, item_all


# speedup vs baseline: 8.4693x; 1.0513x over previous
import functools

import jax
import jax.numpy as jnp
from jax import lax
from jax.experimental import pallas as pl
from jax.experimental.pallas import tpu as pltpu
from jax.experimental.pallas import tpu_sc as plsc

_N_USERS = 5000
_N = 10000
_D = 128
_NNZ = 320000
_NC = 2
_NS = 16
_NW = _NC * _NS
_CHUNK = 80
_TOTC = 4096
_N0 = 128
_N1 = _TOTC // _NS - _N0
_LANES = 16
_G = _D // _LANES
_ZROWS = 640


def _spmm_sc(rows2, cols2, vals2, table):
    mesh = plsc.VectorSubcoreMesh(core_axis_name="c", subcore_axis_name="s")

    @functools.partial(
        pl.kernel,
        out_type=jax.ShapeDtypeStruct((_NC, _N, _D), jnp.float32),
        mesh=mesh,
        scratch_types=dict(
            rows_b=pltpu.VMEM((4, _CHUNK), jnp.int32),
            cols_b=pltpu.VMEM((4, _CHUNK), jnp.int32),
            vals_b=pltpu.VMEM((4, _CHUNK), jnp.float32),
            gb0=pltpu.VMEM((_CHUNK, _D), jnp.float32),
            gb1=pltpu.VMEM((_CHUNK, _D), jnp.float32),
            sb0=pltpu.VMEM((_CHUNK, _D), jnp.float32),
            sb1=pltpu.VMEM((_CHUNK, _D), jnp.float32),
            acc=pltpu.VMEM_SHARED((_N, _D), jnp.float32),
            esem0=pltpu.SemaphoreType.DMA,
            esem1=pltpu.SemaphoreType.DMA,
            esem2=pltpu.SemaphoreType.DMA,
            esem3=pltpu.SemaphoreType.DMA,
            gsem0=pltpu.SemaphoreType.DMA,
            gsem1=pltpu.SemaphoreType.DMA,
            ssem0=pltpu.SemaphoreType.DMA,
            ssem1=pltpu.SemaphoreType.DMA,
        ),
    )
    def k(rows_hbm, cols_hbm, vals_hbm, table_hbm, out_hbm, *, rows_b,
          cols_b, vals_b, gb0, gb1, sb0, sb1, acc, esem0, esem1, esem2,
          esem3, gsem0, gsem1, ssem0, ssem1):
        cid = lax.axis_index("c")
        sid = lax.axis_index("s")
        gb = (gb0, gb1)
        sb = (sb0, sb1)
        esem = (esem0, esem1, esem2, esem3)
        gsem = (gsem0, gsem1)
        ssem = (ssem0, ssem1)

        is0 = cid == 0
        base = jnp.where(is0, sid * _N0, _NS * _N0 + sid * _N1)
        nloc = jnp.where(is0, _N0, _N1)

        def zrow(i, carry):
            for j in range(_G):
                sb0[i, pl.ds(j * _LANES, _LANES)] = jnp.zeros(
                    (_LANES,), jnp.float32)
            return carry

        lax.fori_loop(0, _CHUNK, zrow, 0)
        for kk in range(_ZROWS // _CHUNK):
            pltpu.sync_copy(
                sb0, acc.at[pl.ds(sid * 624 + kk * _CHUNK, _CHUNK)])
        plsc.subcore_barrier()

        def start_edges(c, slot):
            pltpu.async_copy(rows_hbm.at[base + c], rows_b.at[slot],
                             esem[slot])
            pltpu.async_copy(cols_hbm.at[base + c], cols_b.at[slot],
                             esem[slot])
            pltpu.async_copy(vals_hbm.at[base + c], vals_b.at[slot],
                             esem[slot])

        def wait_edges(c, slot):
            pltpu.make_async_copy(rows_hbm.at[base + c], rows_b.at[slot],
                                  esem[slot]).wait()
            pltpu.make_async_copy(cols_hbm.at[base + c], cols_b.at[slot],
                                  esem[slot]).wait()
            pltpu.make_async_copy(vals_hbm.at[base + c], vals_b.at[slot],
                                  esem[slot]).wait()

        def start_gather(slot, b):
            pltpu.async_copy(table_hbm.at[cols_b.at[slot]], gb[b], gsem[b])

        def wait_gather(slot, b):
            pltpu.make_async_copy(table_hbm.at[cols_b.at[slot]], gb[b],
                                  gsem[b]).wait()

        def start_scatter(slot, b):
            pltpu.async_copy(sb[b], acc.at[rows_b.at[slot]], ssem[b],
                             add=True)

        def wait_scatter(slot, b):
            pltpu.make_async_copy(sb[b], acc.at[rows_b.at[slot]],
                                  ssem[b]).wait()

        def scale(slot, b):
            @plsc.parallel_loop(0, _CHUNK // _LANES)
            def _(ii):
                i0 = ii * _LANES
                vv = vals_b[slot, pl.ds(i0, _LANES)]
                for e in range(_LANES):
                    vbc = jnp.broadcast_to(vv[e], (_LANES,))
                    for j in range(_G):
                        sl = pl.ds(j * _LANES, _LANES)
                        sb[b][i0 + e, sl] = gb[b][i0 + e, sl] * vbc

        for c in range(4):
            start_edges(c, c)
        wait_edges(0, 0)
        start_gather(0, 0)
        for m in range(2):
            wait_edges(m + 1, (m + 1) % 4)
            start_gather((m + 1) % 4, (m + 1) % 2)
            wait_gather(m % 4, m % 2)
            scale(m % 4, m % 2)
            start_scatter(m % 4, m % 2)

        def outer(o, carry):
            for b in range(4):
                m = 2 + o * 4 + b
                slot = (2 + b) % 4
                buf = b % 2
                wait_edges(m + 1, (slot + 1) % 4)
                start_gather((slot + 1) % 4, 1 - buf)
                wait_gather(slot, buf)
                wait_scatter(slot, buf)
                start_edges(m + 2, (slot + 2) % 4)
                scale(slot, buf)
                start_scatter(slot, buf)
            return carry

        lax.fori_loop(0, (nloc - 4) // 4, outer, 0)

        m0 = nloc - 2
        wait_edges(m0 + 1, 3)
        start_gather(3, 1)
        wait_gather(2, 0)
        wait_scatter(2, 0)
        scale(2, 0)
        start_scatter(2, 0)

        wait_gather(3, 1)
        wait_scatter(3, 1)
        scale(3, 1)
        start_scatter(3, 1)

        wait_scatter(2, 0)
        wait_scatter(3, 1)
        plsc.subcore_barrier()

        pltpu.sync_copy(acc.at[pl.ds(sid * 624, _ZROWS)],
                        out_hbm.at[cid, pl.ds(sid * 624, _ZROWS)])

    return k(rows2, cols2, vals2, table)


def _tc_add(a, b):
    def body(a_ref, b_ref, o_ref):
        o_ref[...] = a_ref[...] + b_ref[...]

    blk = pl.BlockSpec((1000, _D), lambda i: (i, 0))
    return pl.pallas_call(
        body,
        grid=(_N // 1000,),
        in_specs=[blk, blk],
        out_specs=blk,
        out_shape=jax.ShapeDtypeStruct((_N, _D), jnp.float32),
    )(a, b)


def _tc_assemble(pu, pi, qu0, qu1, qi0, qi1):
    def body(pu_r, pi_r, qu0_r, qu1_r, qi0_r, qi1_r, ou_r, oi_r):
        ou_r[:, : _D] = pu_r[...]
        ou_r[:, _D:] = qu0_r[...] + qu1_r[...]
        oi_r[:, : _D] = pi_r[...]
        oi_r[:, _D:] = qi0_r[...] + qi1_r[...]

    blk = pl.BlockSpec((1000, _D), lambda i: (i, 0))
    oblk = pl.BlockSpec((1000, 2 * _D), lambda i: (i, 0))
    osds = jax.ShapeDtypeStruct((_N_USERS, 2 * _D), jnp.float32)
    return pl.pallas_call(
        body,
        grid=(_N_USERS // 1000,),
        in_specs=[blk] * 6,
        out_specs=[oblk, oblk],
        out_shape=[osds, osds],
    )(pu, pi, qu0, qu1, qi0, qi1)


def _pad_edges(rows, cols, vals):
    pad = _TOTC * _CHUNK - _NNZ
    spread = (jnp.arange(pad, dtype=jnp.int32) * 37) % _N
    rows = jnp.concatenate([rows, spread]).reshape(_TOTC, _CHUNK)
    cols = jnp.concatenate([cols, spread]).reshape(_TOTC, _CHUNK)
    vals = jnp.pad(vals, (0, pad)).reshape(_TOTC, _CHUNK)
    return rows, cols, vals


def kernel(user_emb_table, item_emb_table, adj_rows, adj_cols, adj_vals,
           neg_rows, neg_cols, neg_vals):
    ego0 = jnp.concatenate([user_emb_table, item_emb_table], axis=0)
    ar, ac, av = _pad_edges(adj_rows, adj_cols, adj_vals)
    nr, nc, nv = _pad_edges(neg_rows, neg_cols, neg_vals)

    p = _spmm_sc(ar, ac, av, ego0)
    prop = _tc_add(p[0], p[1])
    q = _spmm_sc(nr, nc, nv, prop)

    user_all, item_all = _tc_assemble(
        prop[:_N_USERS], prop[_N_USERS:],
        q[0, :_N_USERS], q[1, :_N_USERS],
        q[0, _N_USERS:], q[1, _N_USERS:])
    return user_all, item_all
